# Initial kernel scaffold; baseline (speedup 1.0000x reference)
import jax
import jax.numpy as jnp
from jax.experimental import pallas as pl


def kernel(x, edge_index, edge_weight, W1, b1, g1, beta1, W2, b2, g2, beta2, Wo, bo):
    raise NotImplementedError("write your pallas kernel here")


# speedup vs baseline: 13.9610x; 13.9610x over previous
import functools

import jax
import jax.numpy as jnp
from jax import lax
from jax.experimental import pallas as pl
from jax.experimental.pallas import tpu as pltpu
from jax.experimental.pallas import tpu_sc as plsc

_N = 10000
_D = 128
_E = 320000
_NC = 2
_NS = 16
_NW = _NC * _NS
_EPT = _E // _NW
_CB = 80
_NCHUNK = _EPT // _CB
_RPS_A = 624
_RPS_B = _N - 15 * _RPS_A

_mesh = plsc.VectorSubcoreMesh(core_axis_name="c", subcore_axis_name="s")


@functools.partial(
    pl.kernel,
    mesh=_mesh,
    out_type=jax.ShapeDtypeStruct((_NC, _N), jnp.float32),
    scratch_types=[
        pltpu.VMEM((_EPT,), jnp.int32),
        pltpu.VMEM((_EPT,), jnp.float32),
        pltpu.VMEM((_CB,), jnp.int32),
        pltpu.VMEM_SHARED((_N,), jnp.float32),
    ],
)
def _deg_sc(dst_hbm, ew_hbm, zero_hbm, out_hbm, dst_v, ew_v, dchunk, deg_sh):
    c = lax.axis_index("c")
    s = lax.axis_index("s")
    wid = c * _NS + s

    @pl.when(s == 0)
    def _zero():
        pltpu.sync_copy(zero_hbm, deg_sh)

    pltpu.sync_copy(dst_hbm.at[pl.ds(wid * _EPT, _EPT)], dst_v)
    pltpu.sync_copy(ew_hbm.at[pl.ds(wid * _EPT, _EPT)], ew_v)
    plsc.subcore_barrier()

    def body(i, carry):
        base = i * _CB
        for j in range(_CB // 16):
            dchunk[pl.ds(j * 16, 16)] = dst_v[pl.ds(base + j * 16, 16)]
        pltpu.sync_copy(ew_v.at[pl.ds(base, _CB)], deg_sh.at[dchunk], add=True)
        return carry

    lax.fori_loop(0, _NCHUNK, body, 0)
    plsc.subcore_barrier()

    @pl.when(s == 0)
    def _out():
        pltpu.sync_copy(deg_sh, out_hbm.at[c])


@functools.partial(
    pl.kernel,
    mesh=_mesh,
    out_type=jax.ShapeDtypeStruct((_NC, _N, _D), jnp.float32),
    scratch_types=[
        pltpu.VMEM((_EPT,), jnp.int32),
        pltpu.VMEM((_EPT,), jnp.int32),
        pltpu.VMEM((_EPT,), jnp.float32),
        pltpu.VMEM((_CB,), jnp.int32),
        pltpu.VMEM((_CB,), jnp.int32),
        pltpu.VMEM((_CB, _D), jnp.float32),
        pltpu.VMEM_SHARED((_N, _D), jnp.float32),
        pltpu.SemaphoreType.DMA,
    ],
)
def _spmm_sc(hs_hbm, src_hbm, dst_hbm, ew_hbm, zero_hbm, out_hbm,
             src_v, dst_v, ew_v, schunk, dchunk, rows_v, acc_sh, sem):
    c = lax.axis_index("c")
    s = lax.axis_index("s")
    wid = c * _NS + s

    @pl.when(s < 15)
    def _zero_a():
        off = pl.multiple_of(s * _RPS_A, 8)
        pltpu.sync_copy(zero_hbm.at[pl.ds(off, _RPS_A)],
                        acc_sh.at[pl.ds(off, _RPS_A)])

    @pl.when(s == 15)
    def _zero_b():
        pltpu.sync_copy(zero_hbm.at[pl.ds(15 * _RPS_A, _RPS_B)],
                        acc_sh.at[pl.ds(15 * _RPS_A, _RPS_B)])

    pltpu.sync_copy(src_hbm.at[pl.ds(wid * _EPT, _EPT)], src_v)
    pltpu.sync_copy(dst_hbm.at[pl.ds(wid * _EPT, _EPT)], dst_v)
    pltpu.sync_copy(ew_hbm.at[pl.ds(wid * _EPT, _EPT)], ew_v)
    plsc.subcore_barrier()

    def body(i, carry):
        base = i * _CB
        for j in range(_CB // 16):
            schunk[pl.ds(j * 16, 16)] = src_v[pl.ds(base + j * 16, 16)]
            dchunk[pl.ds(j * 16, 16)] = dst_v[pl.ds(base + j * 16, 16)]
        pltpu.async_copy(hs_hbm.at[schunk], rows_v, sem).wait()

        def scale(g, cc):
            ew16 = ew_v[pl.ds(base + g * 16, 16)]
            for e in range(16):
                ewb = jnp.full((16,), ew16[e], jnp.float32)
                r = g * 16 + e
                for j in range(_D // 16):
                    rows_v[r, pl.ds(j * 16, 16)] = \
                        rows_v[r, pl.ds(j * 16, 16)] * ewb
            return cc

        lax.fori_loop(0, _CB // 16, scale, 0)
        pltpu.sync_copy(rows_v, acc_sh.at[dchunk], add=True)
        return carry

    lax.fori_loop(0, _NCHUNK, body, 0)
    plsc.subcore_barrier()

    @pl.when(s < 15)
    def _out_a():
        off = pl.multiple_of(s * _RPS_A, 8)
        pltpu.sync_copy(acc_sh.at[pl.ds(off, _RPS_A)],
                        out_hbm.at[c, pl.ds(off, _RPS_A)])

    @pl.when(s == 15)
    def _out_b():
        pltpu.sync_copy(acc_sh.at[pl.ds(15 * _RPS_A, _RPS_B)],
                        out_hbm.at[c, pl.ds(15 * _RPS_A, _RPS_B)])


def _stage1_body(deg_ref, x_ref, w_ref, dis_ref, hs_ref):
    deg = deg_ref[0, :] + deg_ref[1, :] + 1.0
    dis = lax.rsqrt(deg)
    dis_ref[...] = dis
    h = jnp.dot(x_ref[...], w_ref[...], preferred_element_type=jnp.float32)
    hs_ref[...] = h * dis[:, None]


_stage1 = pl.pallas_call(
    _stage1_body,
    out_shape=(jax.ShapeDtypeStruct((_N,), jnp.float32),
               jax.ShapeDtypeStruct((_N, _D), jnp.float32)),
)


def _stage2_body(acc_ref, hs_ref, dis_ref, b_ref, g_ref, bt_ref, w_ref,
                 h_ref, hs2_ref):
    dis = dis_ref[...]
    t = (acc_ref[0] + acc_ref[1] + hs_ref[...]) * dis[:, None] + b_ref[...][None, :]
    mu = jnp.mean(t, axis=0)
    var = jnp.mean((t - mu[None, :]) ** 2, axis=0)
    hh = g_ref[...][None, :] * (t - mu[None, :]) / jnp.sqrt(var + 1e-5)[None, :] \
        + bt_ref[...][None, :]
    hh = jnp.maximum(hh, 0.0)
    h_ref[...] = hh
    hs2_ref[...] = jnp.dot(hh, w_ref[...], preferred_element_type=jnp.float32) \
        * dis[:, None]


_stage2 = pl.pallas_call(
    _stage2_body,
    out_shape=(jax.ShapeDtypeStruct((_N, _D), jnp.float32),
               jax.ShapeDtypeStruct((_N, _D), jnp.float32)),
)


def _stage2b_body(acc_ref, hs_ref, dis_ref, b_ref, g_ref, bt_ref, w_ref,
                  hp_ref, hs3_ref):
    dis = dis_ref[...]
    t = (acc_ref[0] + acc_ref[1] + hs_ref[...]) * dis[:, None] + b_ref[...][None, :]
    mu = jnp.mean(t, axis=0)
    var = jnp.mean((t - mu[None, :]) ** 2, axis=0)
    hh = g_ref[...][None, :] * (t - mu[None, :]) / jnp.sqrt(var + 1e-5)[None, :] \
        + bt_ref[...][None, :]
    hh = jnp.maximum(hh, 0.0) + hp_ref[...]
    hs3_ref[...] = jnp.dot(hh, w_ref[...], preferred_element_type=jnp.float32) \
        * dis[:, None]


_stage2b = pl.pallas_call(
    _stage2b_body,
    out_shape=jax.ShapeDtypeStruct((_N, _D), jnp.float32),
)


def _stage3_body(acc_ref, hs_ref, dis_ref, bo_ref, out_ref):
    out_ref[...] = (acc_ref[0] + acc_ref[1] + hs_ref[...]) \
        * dis_ref[...][:, None] + bo_ref[...][None, :]


_stage3 = pl.pallas_call(
    _stage3_body,
    out_shape=jax.ShapeDtypeStruct((_N, _D), jnp.float32),
)


def kernel(x, edge_index, edge_weight, W1, b1, g1, beta1, W2, b2, g2, beta2,
           Wo, bo):
    src = edge_index[0].astype(jnp.int32)
    dst = edge_index[1].astype(jnp.int32)
    ew = edge_weight.astype(jnp.float32)
    zero1 = jnp.zeros((_N,), jnp.float32)
    zero2 = jnp.zeros((_N, _D), jnp.float32)

    deg2 = _deg_sc(dst, ew, zero1)
    dis, hs1 = _stage1(deg2, x, W1)
    acc1 = _spmm_sc(hs1, src, dst, ew, zero2)
    h1, hs2 = _stage2(acc1, hs1, dis, b1, g1, beta1, W2)
    acc2 = _spmm_sc(hs2, src, dst, ew, zero2)
    hs3 = _stage2b(acc2, hs2, dis, b2, g2, beta2, Wo, h1)
    acc3 = _spmm_sc(hs3, src, dst, ew, zero2)
    return _stage3(acc3, hs3, dis, bo)


# speedup vs baseline: 23.5429x; 1.6863x over previous
import functools

import jax
import jax.numpy as jnp
from jax import lax
from jax.experimental import pallas as pl
from jax.experimental.pallas import tpu as pltpu
from jax.experimental.pallas import tpu_sc as plsc

_N = 10000
_D = 128
_E = 320000
_NC = 2
_NS = 16
_NW = _NC * _NS
_EPT = _E // _NW
_CB = 80
_NCHUNK = _EPT // _CB
_RPS_A = 624
_RPS_B = _N - 15 * _RPS_A

_mesh = plsc.VectorSubcoreMesh(core_axis_name="c", subcore_axis_name="s")


@functools.partial(
    pl.kernel,
    mesh=_mesh,
    out_type=jax.ShapeDtypeStruct((_NC, _N), jnp.float32),
    scratch_types=[
        pltpu.VMEM((_EPT,), jnp.int32),
        pltpu.VMEM((_EPT,), jnp.float32),
        pltpu.VMEM((_CB,), jnp.int32),
        pltpu.VMEM_SHARED((_N,), jnp.float32),
    ],
)
def _deg_sc(dst_hbm, ew_hbm, zero_hbm, out_hbm, dst_v, ew_v, dchunk, deg_sh):
    c = lax.axis_index("c")
    s = lax.axis_index("s")
    wid = c * _NS + s

    @pl.when(s == 0)
    def _zero():
        pltpu.sync_copy(zero_hbm, deg_sh)

    pltpu.sync_copy(dst_hbm.at[pl.ds(wid * _EPT, _EPT)], dst_v)
    pltpu.sync_copy(ew_hbm.at[pl.ds(wid * _EPT, _EPT)], ew_v)
    plsc.subcore_barrier()

    def body(i, carry):
        base = i * _CB
        for j in range(_CB // 16):
            dchunk[pl.ds(j * 16, 16)] = dst_v[pl.ds(base + j * 16, 16)]
        pltpu.sync_copy(ew_v.at[pl.ds(base, _CB)], deg_sh.at[dchunk], add=True)
        return carry

    lax.fori_loop(0, _NCHUNK, body, 0)
    plsc.subcore_barrier()

    @pl.when(s == 0)
    def _out():
        pltpu.sync_copy(deg_sh, out_hbm.at[c])


@functools.partial(
    pl.kernel,
    mesh=_mesh,
    out_type=jax.ShapeDtypeStruct((_NC, _N, _D), jnp.float32),
    scratch_types=[
        pltpu.VMEM((_EPT,), jnp.int32),
        pltpu.VMEM((_EPT,), jnp.float32),
        pltpu.VMEM((_CB,), jnp.int32),
        pltpu.VMEM((_CB,), jnp.int32),
        pltpu.VMEM((_CB, _D), jnp.float32),
        pltpu.VMEM((_CB, _D), jnp.float32),
        pltpu.VMEM_SHARED((_N, _D), jnp.float32),
        pltpu.SemaphoreType.DMA,
        pltpu.SemaphoreType.DMA,
        pltpu.SemaphoreType.DMA,
        pltpu.SemaphoreType.DMA,
    ],
)
def _spmm_sc(hs_hbm, src_hbm, dst_hbm, ew_hbm, zero_hbm, out_hbm,
             src_v, ew_v, dch0, dch1, rows0, rows1,
             acc_sh, sem0, sem1, dsem0, dsem1):
    c = lax.axis_index("c")
    s = lax.axis_index("s")
    wid = c * _NS + s

    @pl.when(s < 15)
    def _zero_a():
        off = pl.multiple_of(s * _RPS_A, 8)
        pltpu.sync_copy(zero_hbm.at[pl.ds(off, _RPS_A)],
                        acc_sh.at[pl.ds(off, _RPS_A)])

    @pl.when(s == 15)
    def _zero_b():
        pltpu.sync_copy(zero_hbm.at[pl.ds(15 * _RPS_A, _RPS_B)],
                        acc_sh.at[pl.ds(15 * _RPS_A, _RPS_B)])

    pltpu.sync_copy(src_hbm.at[pl.ds(wid * _EPT, _EPT)], src_v)
    pltpu.sync_copy(ew_hbm.at[pl.ds(wid * _EPT, _EPT)], ew_v)
    plsc.subcore_barrier()

    def _dst_desc(ci, dch, dsem):
        off = pl.multiple_of(wid * _EPT + ci * _CB, 8)
        return pltpu.make_async_copy(dst_hbm.at[pl.ds(off, _CB)], dch, dsem)

    def _gather_desc(ci, rows, sem):
        base = pl.multiple_of(ci * _CB, 8)
        return pltpu.make_async_copy(hs_hbm.at[src_v.at[pl.ds(base, _CB)]],
                                     rows, sem)

    def dst_dma(ci, dch, dsem):
        _dst_desc(ci, dch, dsem).start()

    def dst_wait(ci, dch, dsem):
        _dst_desc(ci, dch, dsem).wait()

    def gather(ci, rows, sem):
        _gather_desc(ci, rows, sem).start()

    def gather_wait(ci, rows, sem):
        _gather_desc(ci, rows, sem).wait()

    def scale_rows(ci, rows):
        base = ci * _CB

        def scale(g, cc):
            ew16 = ew_v[pl.ds(base + g * 16, 16)]
            for e in range(16):
                ewb = jnp.full((16,), ew16[e], jnp.float32)
                r = g * 16 + e
                for j in range(_D // 16):
                    rows[r, pl.ds(j * 16, 16)] = \
                        rows[r, pl.ds(j * 16, 16)] * ewb
            return cc

        lax.fori_loop(0, _CB // 16, scale, 0)

    dst_dma(0, dch0, dsem0)
    gather(0, rows0, sem0)

    def pair(i, carry):
        c0 = 2 * i
        dst_dma(c0 + 1, dch1, dsem1)
        gather(c0 + 1, rows1, sem1)
        gather_wait(c0, rows0, sem0)
        scale_rows(c0, rows0)
        dst_wait(c0, dch0, dsem0)
        pltpu.sync_copy(rows0, acc_sh.at[dch0], add=True)
        dst_dma(c0 + 2, dch0, dsem0)
        gather(c0 + 2, rows0, sem0)
        gather_wait(c0 + 1, rows1, sem1)
        scale_rows(c0 + 1, rows1)
        dst_wait(c0 + 1, dch1, dsem1)
        pltpu.sync_copy(rows1, acc_sh.at[dch1], add=True)
        return carry

    lax.fori_loop(0, (_NCHUNK - 1) // 2, pair, 0)
    gather_wait(_NCHUNK - 1, rows0, sem0)
    scale_rows(_NCHUNK - 1, rows0)
    dst_wait(_NCHUNK - 1, dch0, dsem0)
    pltpu.sync_copy(rows0, acc_sh.at[dch0], add=True)
    plsc.subcore_barrier()

    @pl.when(s < 15)
    def _out_a():
        off = pl.multiple_of(s * _RPS_A, 8)
        pltpu.sync_copy(acc_sh.at[pl.ds(off, _RPS_A)],
                        out_hbm.at[c, pl.ds(off, _RPS_A)])

    @pl.when(s == 15)
    def _out_b():
        pltpu.sync_copy(acc_sh.at[pl.ds(15 * _RPS_A, _RPS_B)],
                        out_hbm.at[c, pl.ds(15 * _RPS_A, _RPS_B)])


def _stage1_body(deg_ref, x_ref, w_ref, dis_ref, hs_ref):
    deg = deg_ref[0, :] + deg_ref[1, :] + 1.0
    dis = lax.rsqrt(deg)
    dis_ref[...] = dis
    h = jnp.dot(x_ref[...], w_ref[...], preferred_element_type=jnp.float32)
    hs_ref[...] = h * dis[:, None]


_stage1 = pl.pallas_call(
    _stage1_body,
    out_shape=(jax.ShapeDtypeStruct((_N,), jnp.float32),
               jax.ShapeDtypeStruct((_N, _D), jnp.float32)),
)


def _stage2_body(acc_ref, hs_ref, dis_ref, b_ref, g_ref, bt_ref, w_ref,
                 h_ref, hs2_ref):
    dis = dis_ref[...]
    t = (acc_ref[0] + acc_ref[1] + hs_ref[...]) * dis[:, None] + b_ref[...][None, :]
    mu = jnp.mean(t, axis=0)
    var = jnp.mean((t - mu[None, :]) ** 2, axis=0)
    hh = g_ref[...][None, :] * (t - mu[None, :]) / jnp.sqrt(var + 1e-5)[None, :] \
        + bt_ref[...][None, :]
    hh = jnp.maximum(hh, 0.0)
    h_ref[...] = hh
    hs2_ref[...] = jnp.dot(hh, w_ref[...], preferred_element_type=jnp.float32) \
        * dis[:, None]


_stage2 = pl.pallas_call(
    _stage2_body,
    out_shape=(jax.ShapeDtypeStruct((_N, _D), jnp.float32),
               jax.ShapeDtypeStruct((_N, _D), jnp.float32)),
)


def _stage2b_body(acc_ref, hs_ref, dis_ref, b_ref, g_ref, bt_ref, w_ref,
                  hp_ref, hs3_ref):
    dis = dis_ref[...]
    t = (acc_ref[0] + acc_ref[1] + hs_ref[...]) * dis[:, None] + b_ref[...][None, :]
    mu = jnp.mean(t, axis=0)
    var = jnp.mean((t - mu[None, :]) ** 2, axis=0)
    hh = g_ref[...][None, :] * (t - mu[None, :]) / jnp.sqrt(var + 1e-5)[None, :] \
        + bt_ref[...][None, :]
    hh = jnp.maximum(hh, 0.0) + hp_ref[...]
    hs3_ref[...] = jnp.dot(hh, w_ref[...], preferred_element_type=jnp.float32) \
        * dis[:, None]


_stage2b = pl.pallas_call(
    _stage2b_body,
    out_shape=jax.ShapeDtypeStruct((_N, _D), jnp.float32),
)


def _stage3_body(acc_ref, hs_ref, dis_ref, bo_ref, out_ref):
    out_ref[...] = (acc_ref[0] + acc_ref[1] + hs_ref[...]) \
        * dis_ref[...][:, None] + bo_ref[...][None, :]


_stage3 = pl.pallas_call(
    _stage3_body,
    out_shape=jax.ShapeDtypeStruct((_N, _D), jnp.float32),
)


def kernel(x, edge_index, edge_weight, W1, b1, g1, beta1, W2, b2, g2, beta2,
           Wo, bo):
    src = edge_index[0].astype(jnp.int32)
    dst = edge_index[1].astype(jnp.int32)
    ew = edge_weight.astype(jnp.float32)
    zero1 = jnp.zeros((_N,), jnp.float32)
    zero2 = jnp.zeros((_N, _D), jnp.float32)

    deg2 = _deg_sc(dst, ew, zero1)
    dis, hs1 = _stage1(deg2, x, W1)
    acc1 = _spmm_sc(hs1, src, dst, ew, zero2)
    h1, hs2 = _stage2(acc1, hs1, dis, b1, g1, beta1, W2)
    acc2 = _spmm_sc(hs2, src, dst, ew, zero2)
    hs3 = _stage2b(acc2, hs2, dis, b2, g2, beta2, Wo, h1)
    acc3 = _spmm_sc(hs3, src, dst, ew, zero2)
    return _stage3(acc3, hs3, dis, bo)


# speedup vs baseline: 26.5235x; 1.1266x over previous
import functools

import jax
import jax.numpy as jnp
from jax import lax
from jax.experimental import pallas as pl
from jax.experimental.pallas import tpu as pltpu
from jax.experimental.pallas import tpu_sc as plsc

_N = 10000
_D = 128
_E = 320000
_NC = 2
_NS = 16
_NW = _NC * _NS
_EPT = _E // _NW
_CB = 80
_NCHUNK = _EPT // _CB
_RPS_A = 624
_RPS_B = _N - 15 * _RPS_A

_mesh = plsc.VectorSubcoreMesh(core_axis_name="c", subcore_axis_name="s")


@functools.partial(
    pl.kernel,
    mesh=_mesh,
    out_type=jax.ShapeDtypeStruct((_NC, _N), jnp.float32),
    scratch_types=[
        pltpu.VMEM((_EPT,), jnp.int32),
        pltpu.VMEM((_EPT,), jnp.float32),
        pltpu.VMEM((_CB,), jnp.int32),
        pltpu.VMEM_SHARED((_N,), jnp.float32),
    ],
)
def _deg_sc(dst_hbm, ew_hbm, zero_hbm, out_hbm, dst_v, ew_v, dchunk, deg_sh):
    c = lax.axis_index("c")
    s = lax.axis_index("s")
    wid = c * _NS + s

    @pl.when(s == 0)
    def _zero():
        pltpu.sync_copy(zero_hbm, deg_sh)

    pltpu.sync_copy(dst_hbm.at[pl.ds(wid * _EPT, _EPT)], dst_v)
    pltpu.sync_copy(ew_hbm.at[pl.ds(wid * _EPT, _EPT)], ew_v)
    plsc.subcore_barrier()

    def body(i, carry):
        base = i * _CB
        for j in range(_CB // 16):
            dchunk[pl.ds(j * 16, 16)] = dst_v[pl.ds(base + j * 16, 16)]
        pltpu.sync_copy(ew_v.at[pl.ds(base, _CB)], deg_sh.at[dchunk], add=True)
        return carry

    lax.fori_loop(0, _NCHUNK, body, 0)
    plsc.subcore_barrier()

    @pl.when(s == 0)
    def _out():
        pltpu.sync_copy(deg_sh, out_hbm.at[c])


@functools.partial(
    pl.kernel,
    mesh=_mesh,
    out_type=jax.ShapeDtypeStruct((_NC, _N, _D), jnp.float32),
    scratch_types=[
        pltpu.VMEM((_EPT,), jnp.int32),
        pltpu.VMEM((_CB,), jnp.int32),
        pltpu.VMEM((_CB,), jnp.int32),
        pltpu.VMEM((_CB,), jnp.int32),
        pltpu.VMEM((_CB,), jnp.float32),
        pltpu.VMEM((_CB,), jnp.float32),
        pltpu.VMEM((_CB,), jnp.float32),
        pltpu.VMEM((_CB, _D), jnp.float32),
        pltpu.VMEM((_CB, _D), jnp.float32),
        pltpu.VMEM((_CB, _D), jnp.float32),
        pltpu.VMEM_SHARED((_N, _D), jnp.float32),
        pltpu.SemaphoreType.DMA,
        pltpu.SemaphoreType.DMA,
        pltpu.SemaphoreType.DMA,
        pltpu.SemaphoreType.DMA,
        pltpu.SemaphoreType.DMA,
        pltpu.SemaphoreType.DMA,
        pltpu.SemaphoreType.DMA,
        pltpu.SemaphoreType.DMA,
        pltpu.SemaphoreType.DMA,
        pltpu.SemaphoreType.DMA,
        pltpu.SemaphoreType.DMA,
        pltpu.SemaphoreType.DMA,
    ],
)
def _spmm_sc(hs_hbm, src_hbm, dst_hbm, ew_hbm, zero_hbm, out_hbm,
             src_v, dch0, dch1, dch2, ech0, ech1, ech2,
             rows0, rows1, rows2, acc_sh,
             gs0, gs1, gs2, ds0, ds1, ds2, es0, es1, es2, ss0, ss1, ss2):
    c = lax.axis_index("c")
    s = lax.axis_index("s")
    wid = c * _NS + s

    dchs = (dch0, dch1, dch2)
    echs = (ech0, ech1, ech2)
    rows = (rows0, rows1, rows2)
    gsem = (gs0, gs1, gs2)
    dsem = (ds0, ds1, ds2)
    esem = (es0, es1, es2)
    ssem = (ss0, ss1, ss2)

    @pl.when(s < 15)
    def _zero_a():
        off = pl.multiple_of(s * _RPS_A, 8)
        pltpu.sync_copy(zero_hbm.at[pl.ds(off, _RPS_A)],
                        acc_sh.at[pl.ds(off, _RPS_A)])

    @pl.when(s == 15)
    def _zero_b():
        pltpu.sync_copy(zero_hbm.at[pl.ds(15 * _RPS_A, _RPS_B)],
                        acc_sh.at[pl.ds(15 * _RPS_A, _RPS_B)])

    pltpu.sync_copy(src_hbm.at[pl.ds(wid * _EPT, _EPT)], src_v)
    plsc.subcore_barrier()

    def _dst_desc(ci, k):
        off = pl.multiple_of(wid * _EPT + ci * _CB, 8)
        return pltpu.make_async_copy(dst_hbm.at[pl.ds(off, _CB)],
                                     dchs[k], dsem[k])

    def _ew_desc(ci, k):
        off = pl.multiple_of(wid * _EPT + ci * _CB, 8)
        return pltpu.make_async_copy(ew_hbm.at[pl.ds(off, _CB)],
                                     echs[k], esem[k])

    def _gather_desc(ci, k):
        base = pl.multiple_of(ci * _CB, 8)
        return pltpu.make_async_copy(hs_hbm.at[src_v.at[pl.ds(base, _CB)]],
                                     rows[k], gsem[k])

    def _scatter_desc(k):
        return pltpu.make_async_copy(rows[k], acc_sh.at[dchs[k]], ssem[k])

    def prefetch(ci, k):
        _dst_desc(ci, k).start()
        _ew_desc(ci, k).start()
        _gather_desc(ci, k).start()

    def scale_rows(k):
        ech = echs[k]
        rws = rows[k]

        def scale(g, cc):
            ew16 = ech[pl.ds(g * 16, 16)]
            for e in range(16):
                ewb = jnp.full((16,), ew16[e], jnp.float32)
                r = g * 16 + e
                for j in range(_D // 16):
                    rws[r, pl.ds(j * 16, 16)] = \
                        rws[r, pl.ds(j * 16, 16)] * ewb
            return cc

        lax.fori_loop(0, _CB // 16, scale, 0)

    def process(ci, k):
        _gather_desc(ci, k).wait()
        _ew_desc(ci, k).wait()
        scale_rows(k)
        _dst_desc(ci, k).wait()
        _scatter_desc(k).start(add=True)

    prefetch(0, 0)
    prefetch(1, 1)
    process(0, 0)
    prefetch(2, 2)
    process(1, 1)
    _scatter_desc(0).wait()
    prefetch(3, 0)

    def triple(i, carry):
        c0 = 3 * i + 2
        process(c0, 2)
        _scatter_desc(1).wait()
        prefetch(c0 + 2, 1)
        process(c0 + 1, 0)
        _scatter_desc(2).wait()

        @pl.when(c0 + 3 < _NCHUNK)
        def _():
            prefetch(c0 + 3, 2)

        process(c0 + 2, 1)
        _scatter_desc(0).wait()

        @pl.when(c0 + 4 < _NCHUNK)
        def _():
            prefetch(c0 + 4, 0)

        return carry

    lax.fori_loop(0, (_NCHUNK - 2) // 3, triple, 0)
    _scatter_desc(1).wait()
    plsc.subcore_barrier()

    @pl.when(s < 15)
    def _out_a():
        off = pl.multiple_of(s * _RPS_A, 8)
        pltpu.sync_copy(acc_sh.at[pl.ds(off, _RPS_A)],
                        out_hbm.at[c, pl.ds(off, _RPS_A)])

    @pl.when(s == 15)
    def _out_b():
        pltpu.sync_copy(acc_sh.at[pl.ds(15 * _RPS_A, _RPS_B)],
                        out_hbm.at[c, pl.ds(15 * _RPS_A, _RPS_B)])


def _stage1_body(deg_ref, x_ref, w_ref, dis_ref, hs_ref):
    deg = deg_ref[0, :] + deg_ref[1, :] + 1.0
    dis = lax.rsqrt(deg)
    dis_ref[...] = dis
    h = jnp.dot(x_ref[...], w_ref[...], preferred_element_type=jnp.float32)
    hs_ref[...] = h * dis[:, None]


_stage1 = pl.pallas_call(
    _stage1_body,
    out_shape=(jax.ShapeDtypeStruct((_N,), jnp.float32),
               jax.ShapeDtypeStruct((_N, _D), jnp.float32)),
)


def _stage2_body(acc_ref, hs_ref, dis_ref, b_ref, g_ref, bt_ref, w_ref,
                 h_ref, hs2_ref):
    dis = dis_ref[...]
    t = (acc_ref[0] + acc_ref[1] + hs_ref[...]) * dis[:, None] + b_ref[...][None, :]
    mu = jnp.mean(t, axis=0)
    var = jnp.mean(t * t, axis=0) - mu * mu
    hh = g_ref[...][None, :] * (t - mu[None, :]) / jnp.sqrt(var + 1e-5)[None, :] \
        + bt_ref[...][None, :]
    hh = jnp.maximum(hh, 0.0)
    h_ref[...] = hh
    hs2_ref[...] = jnp.dot(hh, w_ref[...], preferred_element_type=jnp.float32) \
        * dis[:, None]


_stage2 = pl.pallas_call(
    _stage2_body,
    out_shape=(jax.ShapeDtypeStruct((_N, _D), jnp.float32),
               jax.ShapeDtypeStruct((_N, _D), jnp.float32)),
)


def _stage2b_body(acc_ref, hs_ref, dis_ref, b_ref, g_ref, bt_ref, w_ref,
                  hp_ref, hs3_ref):
    dis = dis_ref[...]
    t = (acc_ref[0] + acc_ref[1] + hs_ref[...]) * dis[:, None] + b_ref[...][None, :]
    mu = jnp.mean(t, axis=0)
    var = jnp.mean(t * t, axis=0) - mu * mu
    hh = g_ref[...][None, :] * (t - mu[None, :]) / jnp.sqrt(var + 1e-5)[None, :] \
        + bt_ref[...][None, :]
    hh = jnp.maximum(hh, 0.0) + hp_ref[...]
    hs3_ref[...] = jnp.dot(hh, w_ref[...], preferred_element_type=jnp.float32) \
        * dis[:, None]


_stage2b = pl.pallas_call(
    _stage2b_body,
    out_shape=jax.ShapeDtypeStruct((_N, _D), jnp.float32),
)


def _stage3_body(acc_ref, hs_ref, dis_ref, bo_ref, out_ref):
    out_ref[...] = (acc_ref[0] + acc_ref[1] + hs_ref[...]) \
        * dis_ref[...][:, None] + bo_ref[...][None, :]


_stage3 = pl.pallas_call(
    _stage3_body,
    out_shape=jax.ShapeDtypeStruct((_N, _D), jnp.float32),
)


def kernel(x, edge_index, edge_weight, W1, b1, g1, beta1, W2, b2, g2, beta2,
           Wo, bo):
    src = edge_index[0].astype(jnp.int32)
    dst = edge_index[1].astype(jnp.int32)
    ew = edge_weight.astype(jnp.float32)
    zero1 = jnp.zeros((_N,), jnp.float32)
    zero2 = jnp.zeros((_N, _D), jnp.float32)

    deg2 = _deg_sc(dst, ew, zero1)
    dis, hs1 = _stage1(deg2, x, W1)
    acc1 = _spmm_sc(hs1, src, dst, ew, zero2)
    h1, hs2 = _stage2(acc1, hs1, dis, b1, g1, beta1, W2)
    acc2 = _spmm_sc(hs2, src, dst, ew, zero2)
    hs3 = _stage2b(acc2, hs2, dis, b2, g2, beta2, Wo, h1)
    acc3 = _spmm_sc(hs3, src, dst, ew, zero2)
    return _stage3(acc3, hs3, dis, bo)
